# baseline (speedup 1.0000x reference)
import functools

import jax
import jax.numpy as jnp
from jax import lax
from jax.experimental import pallas as pl
from jax.experimental.pallas import tpu as pltpu
from jax.experimental.pallas import tpu_sc as plsc

VOCAB = 1_000_000
EMB = 64
NTOK = 819_200

_info = plsc.get_sparse_core_info()
_NC = _info.num_cores
_NS = _info.num_subcores
NW = _NC * _NS
B_PER_W = NTOK // NW
CHUNK = 128
N_CHUNKS = B_PER_W // CHUNK
NBUF = 4

_mesh = plsc.VectorSubcoreMesh(core_axis_name="c", subcore_axis_name="s")


@functools.partial(
    pl.kernel,
    mesh=_mesh,
    out_type=jax.ShapeDtypeStruct((NTOK, 2 * EMB), jnp.float32),
    scratch_types=[
        pltpu.VMEM((B_PER_W,), jnp.int32),
        pltpu.VMEM((NBUF, CHUNK, 2 * EMB), jnp.float32),
    ] + [pltpu.SemaphoreType.DMA] * NBUF,
    compiler_params=pltpu.CompilerParams(use_tc_tiling_on_sc=False),
)
def _gather_kernel(ids_hbm, table_hbm, out_hbm, idx_v, rows_v, *gsems):
    wid = lax.axis_index("s") * _NC + lax.axis_index("c")
    base = wid * B_PER_W

    pltpu.sync_copy(ids_hbm.at[pl.ds(base, B_PER_W)], idx_v)

    def gather(c, s):
        return pltpu.make_async_copy(
            table_hbm.at[idx_v.at[pl.ds(c * CHUNK, CHUNK)]],
            rows_v.at[s],
            gsems[s],
        )

    for s in range(NBUF):
        gather(s, s).start()

    def round_step(r, _):
        for s in range(NBUF):
            g = r * NBUF + s
            gather(g, s).wait()
            pltpu.sync_copy(
                rows_v.at[s, :, pl.ds(0, EMB)],
                out_hbm.at[pl.ds(base + g * CHUNK, CHUNK), pl.ds(0, EMB)],
            )

            @pl.when(g + NBUF < N_CHUNKS)
            def _():
                gather(g + NBUF, s).start()

        return 0

    lax.fori_loop(0, N_CHUNKS // NBUF, round_step, 0)


def kernel(tokens_ids, table):
    table_wide = jnp.pad(table, ((0, 0), (0, EMB)))
    out_wide = _gather_kernel(tokens_ids.astype(jnp.int32), table_wide)
    return out_wide[:, :EMB]


# speedup vs baseline: 1.0048x; 1.0048x over previous
import functools

import jax
import jax.numpy as jnp
from jax import lax
from jax.experimental import pallas as pl
from jax.experimental.pallas import tpu as pltpu
from jax.experimental.pallas import tpu_sc as plsc

VOCAB = 1_000_000
EMB = 64
NTOK = 819_200

_info = plsc.get_sparse_core_info()
_NC = _info.num_cores
_NS = _info.num_subcores
NW = _NC * _NS
B_PER_W = NTOK // NW
CHUNK = 128
N_CHUNKS = B_PER_W // CHUNK
NBUF = 8

_mesh = plsc.VectorSubcoreMesh(core_axis_name="c", subcore_axis_name="s")


@functools.partial(
    pl.kernel,
    mesh=_mesh,
    out_type=jax.ShapeDtypeStruct((NTOK, 2 * EMB), jnp.float32),
    scratch_types=[
        pltpu.VMEM((B_PER_W,), jnp.int32),
        pltpu.VMEM((NBUF, CHUNK, EMB), jnp.float32),
    ] + [pltpu.SemaphoreType.DMA] * NBUF,
    compiler_params=pltpu.CompilerParams(use_tc_tiling_on_sc=False),
)
def _gather_kernel(ids_hbm, table_hbm, out_hbm, idx_v, rows_v, *gsems):
    wid = lax.axis_index("s") * _NC + lax.axis_index("c")
    base = wid * B_PER_W

    pltpu.sync_copy(ids_hbm.at[pl.ds(base, B_PER_W)], idx_v)

    def gather(c, s):
        return pltpu.make_async_copy(
            table_hbm.at[idx_v.at[pl.ds(c * CHUNK, CHUNK)]],
            rows_v.at[s],
            gsems[s],
        )

    for s in range(NBUF):
        gather(s, s).start()

    def round_step(r, _):
        for s in range(NBUF):
            g = r * NBUF + s
            gather(g, s).wait()
            pltpu.sync_copy(
                rows_v.at[s],
                out_hbm.at[pl.ds(base + g * CHUNK, CHUNK), pl.ds(0, EMB)],
            )

            @pl.when(g + NBUF < N_CHUNKS)
            def _():
                gather(g + NBUF, s).start()

        return 0

    lax.fori_loop(0, N_CHUNKS // NBUF, round_step, 0)


def kernel(tokens_ids, table):
    out_wide = _gather_kernel(tokens_ids.astype(jnp.int32), table)
    return out_wide[:, :EMB]
